# baseline (speedup 1.0000x reference)
import functools

import jax
import jax.numpy as jnp
from jax import lax
from jax.experimental import pallas as pl
from jax.experimental.pallas import tpu as pltpu
from jax.experimental.pallas import tpu_sc as plsc

N = 10000
NP = 10240
E = 320000
D = 128
NC = 2
NS = 16
CH = 128

_EC = E // NC
_STRIDE = _EC // CH
_NCHUNK = _STRIDE // NS
_NPAIR = _NCHUNK // 2
_NEXTRA = _STRIDE - NS * _NCHUNK
_RPT = NP // NS
_ZROWS = 128
_CZ = 2048


def _sc_partials(f0, r0, f1, r1, f2, r2):
    mesh = plsc.VectorSubcoreMesh(core_axis_name="c", subcore_axis_name="s")

    @functools.partial(
        pl.kernel,
        mesh=mesh,
        out_type=[
            jax.ShapeDtypeStruct((NC * 3 * NP, D), jnp.float32),
            jax.ShapeDtypeStruct((NC * 3 * NP,), jnp.float32),
        ],
        scratch_types=[
            pltpu.VMEM((CH,), jnp.int32),
            pltpu.VMEM((CH,), jnp.int32),
            pltpu.VMEM((CH, D), jnp.float32),
            pltpu.VMEM((CH, D), jnp.float32),
            pltpu.VMEM((CH,), jnp.float32),
            pltpu.VMEM((CH,), jnp.int32),
            pltpu.VMEM((CH,), jnp.int32),
            pltpu.VMEM_SHARED((NP, D), jnp.float32),
            pltpu.VMEM_SHARED((NP,), jnp.float32),
            pltpu.SemaphoreType.DMA,
            pltpu.SemaphoreType.DMA,
            pltpu.SemaphoreType.DMA,
            pltpu.SemaphoreType.DMA,
        ],
    )
    def k(f0h, r0h, f1h, r1h, f2h, r2h, ixh, z2dh, z1dh, onesh,
          sums_out, cnts_out,
          rvA, rvB, fvA, fvB, ones_v, ixA, ixB, acc, cnt,
          gA, gB, sA, sB):
        c = lax.axis_index("c")
        s = lax.axis_index("s")
        pltpu.sync_copy(onesh, ones_v)
        feats = (f0h, f1h, f2h)
        recvs = (r0h, r1h, r2h)

        def load_small(rh, q, ix, rv):
            off = (c * _STRIDE + q) * CH
            pltpu.sync_copy(ixh.at[pl.ds(off, CH)], ix)
            pltpu.sync_copy(rh.at[pl.ds(off, CH)], rv)

        def zero_acc():
            for z in range(_RPT // _ZROWS):
                pltpu.sync_copy(z2dh, acc.at[pl.ds(s * _RPT + z * _ZROWS, _ZROWS)])

            @pl.when(s == 0)
            def _():
                for z in range(NP // _CZ):
                    pltpu.sync_copy(z1dh, cnt.at[pl.ds(z * _CZ, _CZ)])

        zero_acc()
        plsc.subcore_barrier()

        for t in range(3):
            fh = feats[t]
            rh = recvs[t]

            def start_g(ix, fv, sem):
                pltpu.async_copy(fh.at[ix], fv, sem)

            def wait_g(ix, fv, sem):
                pltpu.make_async_copy(fh.at[ix], fv, sem).wait()

            def start_s(rv, fv, sem):
                pltpu.async_copy(fv, acc.at[rv], sem, add=True)
                pltpu.async_copy(ones_v, cnt.at[rv], sem, add=True)

            def wait_s(rv, fv, sem):
                pltpu.make_async_copy(fv, acc.at[rv], sem).wait()
                pltpu.make_async_copy(ones_v, cnt.at[rv], sem).wait()

            load_small(rh, s, ixA, rvA)
            start_g(ixA, fvA, gA)

            def pair(p, carry):
                q0 = s + 16 * (2 * p)

                @pl.when(p >= 1)
                def _():
                    wait_s(rvB, fvB, sB)

                load_small(rh, q0 + 16, ixB, rvB)
                start_g(ixB, fvB, gB)
                wait_g(ixA, fvA, gA)
                start_s(rvA, fvA, sA)
                wait_s(rvA, fvA, sA)
                load_small(rh, q0 + 32, ixA, rvA)
                start_g(ixA, fvA, gA)
                wait_g(ixB, fvB, gB)
                start_s(rvB, fvB, sB)
                return carry

            lax.fori_loop(0, _NPAIR - 1, pair, 0)
            wait_s(rvB, fvB, sB)
            load_small(rh, s + 16 * 77, ixB, rvB)
            start_g(ixB, fvB, gB)
            wait_g(ixA, fvA, gA)
            start_s(rvA, fvA, sA)
            wait_s(rvA, fvA, sA)
            wait_g(ixB, fvB, gB)
            start_s(rvB, fvB, sB)
            wait_s(rvB, fvB, sB)

            @pl.when(s < _NEXTRA)
            def _():
                load_small(rh, NS * _NCHUNK + s, ixA, rvA)
                pltpu.sync_copy(fh.at[ixA], fvA)
                pltpu.sync_copy(fvA, acc.at[rvA], add=True)
                pltpu.sync_copy(ones_v, cnt.at[rvA], add=True)

            plsc.subcore_barrier()

            off = (c * 3 + t) * NP
            for z in range(_RPT // _ZROWS):
                r0_ = s * _RPT + z * _ZROWS
                pltpu.sync_copy(acc.at[pl.ds(r0_, _ZROWS)],
                                sums_out.at[pl.ds(off + r0_, _ZROWS)])

            @pl.when(s == 0)
            def _():
                pltpu.sync_copy(cnt, cnts_out.at[pl.ds(off, NP)])

            if t < 2:
                zero_acc()
            plsc.subcore_barrier()

    z2d = jnp.zeros((_ZROWS, D), jnp.float32)
    z1d = jnp.zeros((_CZ,), jnp.float32)
    ones = jnp.ones((CH,), jnp.float32)
    ix = (jnp.arange(NC, dtype=jnp.int32)[:, None, None] * _EC
          + jnp.arange(_STRIDE, dtype=jnp.int32)[None, :, None]
          + jnp.arange(CH, dtype=jnp.int32)[None, None, :] * _STRIDE
          ).reshape(-1)
    return k(f0, _perm(r0), f1, _perm(r1), f2, _perm(r2), ix, z2d, z1d, ones)


def _perm(recv):
    return recv.reshape(NC, CH, _STRIDE).transpose(0, 2, 1).reshape(-1)


_BLK = 400


def _merge_body(s_ref, c_ref, o_same, o_anti, o_ee, o_ne):
    s_same = s_ref[0, 0] + s_ref[1, 0]
    s_anti = s_ref[0, 1] + s_ref[1, 1]
    s_ne = s_ref[0, 2] + s_ref[1, 2]
    c_same = c_ref[0, 0] + c_ref[1, 0]
    c_anti = c_ref[0, 1] + c_ref[1, 1]
    c_ne = c_ref[0, 2] + c_ref[1, 2]
    o_same[...] = s_same / jnp.maximum(c_same, 1.0)
    o_anti[...] = s_anti / jnp.maximum(c_anti, 1.0)
    o_ee[...] = (s_same + s_anti) * (1.0 / 10000.0)
    o_ne[...] = s_ne / jnp.maximum(c_ne, 1.0)


def kernel(nodes, feat_same, recv_same, feat_anti, recv_anti, feat_ne, recv_ne):
    del nodes
    sums_flat, cnts_flat = _sc_partials(
        feat_same, recv_same, feat_anti, recv_anti, feat_ne, recv_ne)
    sums4 = sums_flat.reshape(NC, 3, NP, D)
    cnts4 = cnts_flat.reshape(NC, 3, NP, 1)

    outs = pl.pallas_call(
        _merge_body,
        grid=(N // _BLK,),
        in_specs=[
            pl.BlockSpec((NC, 3, _BLK, D), lambda i: (0, 0, i, 0)),
            pl.BlockSpec((NC, 3, _BLK, 1), lambda i: (0, 0, i, 0)),
        ],
        out_specs=[pl.BlockSpec((_BLK, D), lambda i: (i, 0))] * 4,
        out_shape=[jax.ShapeDtypeStruct((N, D), jnp.float32)] * 4,
    )(sums4, cnts4)
    return tuple(outs)


# speedup vs baseline: 1.2887x; 1.2887x over previous
import functools

import jax
import jax.numpy as jnp
from jax import lax
from jax.experimental import pallas as pl
from jax.experimental.pallas import tpu as pltpu
from jax.experimental.pallas import tpu_sc as plsc

N = 10000
NP = 10240
E = 320000
D = 128
NC = 2
NS = 16
CH = 128
CT = 16

_PER_TILE_E = E // (NC * NS)
_NCHUNK = _PER_TILE_E // CH
_NPAIR = 38
_RPT = NP // NS
_ZROWS = 128
_CZ = 2048


def _sc_partials(f0, r0, f1, r1, f2, r2):
    mesh = plsc.VectorSubcoreMesh(core_axis_name="c", subcore_axis_name="s")

    @functools.partial(
        pl.kernel,
        mesh=mesh,
        out_type=[
            jax.ShapeDtypeStruct((NC * 3 * NP, D), jnp.float32),
            jax.ShapeDtypeStruct((NC * 3 * NP,), jnp.float32),
        ],
        scratch_types=[
            pltpu.VMEM((CH,), jnp.int32),
            pltpu.VMEM((CH,), jnp.int32),
            pltpu.VMEM((CH, D), jnp.float32),
            pltpu.VMEM((CH, D), jnp.float32),
            pltpu.VMEM((CH,), jnp.float32),
            pltpu.VMEM((CT,), jnp.int32),
            pltpu.VMEM((CT, D), jnp.float32),
            pltpu.VMEM((CT,), jnp.float32),
            pltpu.VMEM_SHARED((NP, D), jnp.float32),
            pltpu.VMEM_SHARED((NP,), jnp.float32),
            pltpu.SemaphoreType.DMA,
            pltpu.SemaphoreType.DMA,
            pltpu.SemaphoreType.DMA,
            pltpu.SemaphoreType.DMA,
        ],
    )
    def k(f0h, r0h, f1h, r1h, f2h, r2h, z2dh, z1dh, onesh, onesth,
          sums_out, cnts_out,
          rvA, rvB, fvA, fvB, ones_v, rvT, fvT, ones_t, acc, cnt,
          gA, gB, sA, sB):
        c = lax.axis_index("c")
        s = lax.axis_index("s")
        pltpu.sync_copy(onesh, ones_v)
        pltpu.sync_copy(onesth, ones_t)
        feats = (f0h, f1h, f2h)
        recvs = (r0h, r1h, r2h)
        base0 = (c * NS + s) * _PER_TILE_E

        def zero_acc():
            for z in range(_RPT // _ZROWS):
                pltpu.sync_copy(z2dh, acc.at[pl.ds(s * _RPT + z * _ZROWS, _ZROWS)])

            @pl.when(s == 0)
            def _():
                for z in range(NP // _CZ):
                    pltpu.sync_copy(z1dh, cnt.at[pl.ds(z * _CZ, _CZ)])

        zero_acc()
        plsc.subcore_barrier()

        for t in range(3):
            fh = feats[t]
            rh = recvs[t]

            def start_g(base, rv, fv, sem):
                pltpu.async_copy(rh.at[pl.ds(base, CH)], rv, sem)
                pltpu.async_copy(fh.at[pl.ds(base, CH)], fv, sem)

            def wait_g(base, rv, fv, sem):
                pltpu.make_async_copy(rh.at[pl.ds(base, CH)], rv, sem).wait()
                pltpu.make_async_copy(fh.at[pl.ds(base, CH)], fv, sem).wait()

            def start_s(rv, fv, sem):
                pltpu.async_copy(fv, acc.at[rv], sem, add=True)
                pltpu.async_copy(ones_v, cnt.at[rv], sem, add=True)

            def wait_s(rv, fv, sem):
                pltpu.make_async_copy(fv, acc.at[rv], sem).wait()
                pltpu.make_async_copy(ones_v, cnt.at[rv], sem).wait()

            start_g(base0, rvA, fvA, gA)

            def pair(p, carry):
                b0 = base0 + (2 * p) * CH

                @pl.when(p >= 1)
                def _():
                    wait_s(rvB, fvB, sB)

                start_g(b0 + CH, rvB, fvB, gB)
                wait_g(b0, rvA, fvA, gA)
                start_s(rvA, fvA, sA)
                wait_s(rvA, fvA, sA)
                start_g(b0 + 2 * CH, rvA, fvA, gA)
                wait_g(b0 + CH, rvB, fvB, gB)
                start_s(rvB, fvB, sB)
                return carry

            lax.fori_loop(0, _NPAIR, pair, 0)
            wait_s(rvB, fvB, sB)
            start_g(base0 + 77 * CH, rvB, fvB, gB)
            wait_g(base0 + 76 * CH, rvA, fvA, gA)
            start_s(rvA, fvA, sA)
            tbase = base0 + _NCHUNK * CH
            pltpu.async_copy(rh.at[pl.ds(tbase, CT)], rvT, gA)
            pltpu.async_copy(fh.at[pl.ds(tbase, CT)], fvT, gA)
            wait_s(rvA, fvA, sA)
            wait_g(base0 + 77 * CH, rvB, fvB, gB)
            start_s(rvB, fvB, sB)
            wait_s(rvB, fvB, sB)
            pltpu.make_async_copy(rh.at[pl.ds(tbase, CT)], rvT, gA).wait()
            pltpu.make_async_copy(fh.at[pl.ds(tbase, CT)], fvT, gA).wait()
            pltpu.sync_copy(fvT, acc.at[rvT], add=True)
            pltpu.sync_copy(ones_t, cnt.at[rvT], add=True)
            plsc.subcore_barrier()

            off = (c * 3 + t) * NP
            for z in range(_RPT // _ZROWS):
                r0_ = s * _RPT + z * _ZROWS
                pltpu.sync_copy(acc.at[pl.ds(r0_, _ZROWS)],
                                sums_out.at[pl.ds(off + r0_, _ZROWS)])

            @pl.when(s == 0)
            def _():
                pltpu.sync_copy(cnt, cnts_out.at[pl.ds(off, NP)])

            if t < 2:
                zero_acc()
            plsc.subcore_barrier()

    z2d = jnp.zeros((_ZROWS, D), jnp.float32)
    z1d = jnp.zeros((_CZ,), jnp.float32)
    ones = jnp.ones((CH,), jnp.float32)
    onest = jnp.ones((CT,), jnp.float32)
    return k(f0, r0, f1, r1, f2, r2, z2d, z1d, ones, onest)


_BLK = 400


def _merge_body(s_ref, c_ref, o_same, o_anti, o_ee, o_ne):
    s_same = s_ref[0, 0] + s_ref[1, 0]
    s_anti = s_ref[0, 1] + s_ref[1, 1]
    s_ne = s_ref[0, 2] + s_ref[1, 2]
    c_same = c_ref[0, 0] + c_ref[1, 0]
    c_anti = c_ref[0, 1] + c_ref[1, 1]
    c_ne = c_ref[0, 2] + c_ref[1, 2]
    o_same[...] = s_same / jnp.maximum(c_same, 1.0)
    o_anti[...] = s_anti / jnp.maximum(c_anti, 1.0)
    o_ee[...] = (s_same + s_anti) * (1.0 / 10000.0)
    o_ne[...] = s_ne / jnp.maximum(c_ne, 1.0)


def kernel(nodes, feat_same, recv_same, feat_anti, recv_anti, feat_ne, recv_ne):
    del nodes
    sums_flat, cnts_flat = _sc_partials(
        feat_same, recv_same, feat_anti, recv_anti, feat_ne, recv_ne)
    sums4 = sums_flat.reshape(NC, 3, NP, D)
    cnts4 = cnts_flat.reshape(NC, 3, NP, 1)

    outs = pl.pallas_call(
        _merge_body,
        grid=(N // _BLK,),
        in_specs=[
            pl.BlockSpec((NC, 3, _BLK, D), lambda i: (0, 0, i, 0)),
            pl.BlockSpec((NC, 3, _BLK, 1), lambda i: (0, 0, i, 0)),
        ],
        out_specs=[pl.BlockSpec((_BLK, D), lambda i: (i, 0))] * 4,
        out_shape=[jax.ShapeDtypeStruct((N, D), jnp.float32)] * 4,
    )(sums4, cnts4)
    return tuple(outs)


# speedup vs baseline: 1.3091x; 1.0158x over previous
import functools

import jax
import jax.numpy as jnp
from jax import lax
from jax.experimental import pallas as pl
from jax.experimental.pallas import tpu as pltpu
from jax.experimental.pallas import tpu_sc as plsc

N = 10000
NP = 10240
E = 320000
D = 128
NC = 2
NS = 16
CH = 128
CT = 16

_PER_TILE_E = E // (NC * NS)
_NCHUNK = _PER_TILE_E // CH
_NPAIR = 38
_RPT = NP // NS
_ZROWS = 128
_CZ = 2048


def _sc_partials(f0, r0, f1, r1, f2, r2):
    mesh = plsc.VectorSubcoreMesh(core_axis_name="c", subcore_axis_name="s")

    @functools.partial(
        pl.kernel,
        mesh=mesh,
        out_type=[
            jax.ShapeDtypeStruct((NC * 3 * NP, D), jnp.float32),
            jax.ShapeDtypeStruct((NC * 3 * NP,), jnp.float32),
        ],
        scratch_types=[
            pltpu.VMEM((CH,), jnp.int32),
            pltpu.VMEM((CH,), jnp.int32),
            pltpu.VMEM((CH, D), jnp.float32),
            pltpu.VMEM((CH, D), jnp.float32),
            pltpu.VMEM((CH,), jnp.float32),
            pltpu.VMEM((CT,), jnp.int32),
            pltpu.VMEM((CT, D), jnp.float32),
            pltpu.VMEM((CT,), jnp.float32),
            pltpu.VMEM_SHARED((NP, D), jnp.float32),
            pltpu.VMEM_SHARED((NP,), jnp.float32),
            pltpu.SemaphoreType.DMA,
            pltpu.SemaphoreType.DMA,
            pltpu.SemaphoreType.DMA,
            pltpu.SemaphoreType.DMA,
        ],
    )
    def k(f0h, r0h, f1h, r1h, f2h, r2h, z2dh, z1dh, onesh, onesth,
          sums_out, cnts_out,
          rvA, rvB, fvA, fvB, ones_v, rvT, fvT, ones_t, acc, cnt,
          gA, gB, sA, sB):
        c = lax.axis_index("c")
        s = lax.axis_index("s")
        pltpu.sync_copy(onesh, ones_v)
        pltpu.sync_copy(onesth, ones_t)
        feats = (f0h, f1h, f2h)
        recvs = (r0h, r1h, r2h)
        base0 = (c * NS + s) * _PER_TILE_E

        def zero_acc():
            for z in range(_RPT // _ZROWS):
                pltpu.sync_copy(z2dh, acc.at[pl.ds(s * _RPT + z * _ZROWS, _ZROWS)])

            @pl.when(s == 0)
            def _():
                for z in range(NP // _CZ):
                    pltpu.sync_copy(z1dh, cnt.at[pl.ds(z * _CZ, _CZ)])

        zero_acc()
        plsc.subcore_barrier()

        for t in range(3):
            fh = feats[t]
            rh = recvs[t]

            def start_g(base, rv, fv, sem):
                pltpu.async_copy(rh.at[pl.ds(base, CH)], rv, sem)
                pltpu.async_copy(fh.at[pl.ds(base, CH)], fv, sem)

            def wait_g(base, rv, fv, sem):
                pltpu.make_async_copy(rh.at[pl.ds(base, CH)], rv, sem).wait()
                pltpu.make_async_copy(fh.at[pl.ds(base, CH)], fv, sem).wait()

            def start_s(rv, fv, sem):
                pltpu.async_copy(fv, acc.at[rv], sem, add=True)
                pltpu.async_copy(ones_v, cnt.at[rv], sem, add=True)

            def wait_s(rv, fv, sem):
                pltpu.make_async_copy(fv, acc.at[rv], sem).wait()
                pltpu.make_async_copy(ones_v, cnt.at[rv], sem).wait()

            start_g(base0, rvA, fvA, gA)

            def pair(p, carry):
                b0 = base0 + (2 * p) * CH

                @pl.when(p >= 1)
                def _():
                    wait_s(rvB, fvB, sB)

                start_g(b0 + CH, rvB, fvB, gB)
                wait_g(b0, rvA, fvA, gA)
                start_s(rvA, fvA, sA)
                wait_s(rvA, fvA, sA)
                start_g(b0 + 2 * CH, rvA, fvA, gA)
                wait_g(b0 + CH, rvB, fvB, gB)
                start_s(rvB, fvB, sB)
                return carry

            lax.fori_loop(0, _NPAIR, pair, 0)
            wait_s(rvB, fvB, sB)
            start_g(base0 + 77 * CH, rvB, fvB, gB)
            wait_g(base0 + 76 * CH, rvA, fvA, gA)
            start_s(rvA, fvA, sA)
            tbase = base0 + _NCHUNK * CH
            pltpu.async_copy(rh.at[pl.ds(tbase, CT)], rvT, gA)
            pltpu.async_copy(fh.at[pl.ds(tbase, CT)], fvT, gA)
            wait_s(rvA, fvA, sA)
            wait_g(base0 + 77 * CH, rvB, fvB, gB)
            start_s(rvB, fvB, sB)
            wait_s(rvB, fvB, sB)
            pltpu.make_async_copy(rh.at[pl.ds(tbase, CT)], rvT, gA).wait()
            pltpu.make_async_copy(fh.at[pl.ds(tbase, CT)], fvT, gA).wait()
            pltpu.sync_copy(fvT, acc.at[rvT], add=True)
            pltpu.sync_copy(ones_t, cnt.at[rvT], add=True)
            plsc.subcore_barrier()

            off = (c * 3 + t) * NP
            for z in range(_RPT // _ZROWS):
                r0_ = s * _RPT + z * _ZROWS
                pltpu.sync_copy(acc.at[pl.ds(r0_, _ZROWS)],
                                sums_out.at[pl.ds(off + r0_, _ZROWS)])

            @pl.when(s == 0)
            def _():
                pltpu.sync_copy(cnt, cnts_out.at[pl.ds(off, NP)])

            if t < 2:
                zero_acc()
            plsc.subcore_barrier()

    z2d = jnp.zeros((_ZROWS, D), jnp.float32)
    z1d = jnp.zeros((_CZ,), jnp.float32)
    ones = jnp.ones((CH,), jnp.float32)
    onest = jnp.ones((CT,), jnp.float32)
    return k(f0, r0, f1, r1, f2, r2, z2d, z1d, ones, onest)


_BLK = 1000


def _merge_body(s_ref, c_ref, o_same, o_anti, o_ee, o_ne):
    s_same = s_ref[0, 0] + s_ref[1, 0]
    s_anti = s_ref[0, 1] + s_ref[1, 1]
    s_ne = s_ref[0, 2] + s_ref[1, 2]
    c_same = c_ref[0, 0] + c_ref[1, 0]
    c_anti = c_ref[0, 1] + c_ref[1, 1]
    c_ne = c_ref[0, 2] + c_ref[1, 2]
    o_same[...] = s_same / jnp.maximum(c_same, 1.0)
    o_anti[...] = s_anti / jnp.maximum(c_anti, 1.0)
    o_ee[...] = (s_same + s_anti) * (1.0 / 10000.0)
    o_ne[...] = s_ne / jnp.maximum(c_ne, 1.0)


def kernel(nodes, feat_same, recv_same, feat_anti, recv_anti, feat_ne, recv_ne):
    del nodes
    sums_flat, cnts_flat = _sc_partials(
        feat_same, recv_same, feat_anti, recv_anti, feat_ne, recv_ne)
    sums4 = sums_flat.reshape(NC, 3, NP, D)
    cnts4 = cnts_flat.reshape(NC, 3, NP, 1)

    outs = pl.pallas_call(
        _merge_body,
        grid=(N // _BLK,),
        in_specs=[
            pl.BlockSpec((NC, 3, _BLK, D), lambda i: (0, 0, i, 0)),
            pl.BlockSpec((NC, 3, _BLK, 1), lambda i: (0, 0, i, 0)),
        ],
        out_specs=[pl.BlockSpec((_BLK, D), lambda i: (i, 0))] * 4,
        out_shape=[jax.ShapeDtypeStruct((N, D), jnp.float32)] * 4,
    )(sums4, cnts4)
    return tuple(outs)
